# baseline (speedup 1.0000x reference)
import jax
import jax.numpy as jnp
from jax import lax
from jax.experimental import pallas as pl
from jax.experimental.pallas import tpu as pltpu
from jax.experimental.pallas import tpu_sc as plsc

N = 10000
D = 256
B = 1024


def _sc_center_body(idx_hbm, emb_hbm, center_out, idx_v, cen_v, sem):
    nc = 2
    ns = 16
    wid = lax.axis_index("s") * nc + lax.axis_index("c")
    per_w = B // (nc * ns)
    base = wid * per_w
    pltpu.sync_copy(idx_hbm.at[pl.ds(base, per_w)], idx_v)
    pltpu.async_copy(emb_hbm.at[idx_v], cen_v, sem).wait()
    pltpu.sync_copy(cen_v, center_out.at[pl.ds(base, per_w)])


def _sc_center(node_indexes, emb):
    mesh = plsc.VectorSubcoreMesh(core_axis_name="c", subcore_axis_name="s")
    fn = pl.kernel(
        _sc_center_body,
        mesh=mesh,
        out_type=jax.ShapeDtypeStruct((B, D), jnp.float32),
        scratch_types=[
            pltpu.VMEM((B // 32,), jnp.int32),
            pltpu.VMEM((B // 32, D), jnp.float32),
            pltpu.SemaphoreType.DMA,
        ],
        compiler_params=pltpu.CompilerParams(use_tc_tiling_on_sc=False),
    )
    return fn(node_indexes, emb)


_RB = 128
_NBLK = B // _RB


def _tc_body(idx_ref, a_hbm, emb_ref, cen_ref, w1_ref, w2t_ref, b_ref,
             out_ref, a_buf, t_scr, sems):
    i = pl.program_id(0)
    slot = lax.rem(i, 2)

    def issue(block, slot_):
        def row(r, _):
            g = idx_ref[block * _RB + r]
            pltpu.make_async_copy(
                a_hbm.at[pl.ds(g, 1)],
                a_buf.at[slot_, pl.ds(r, 1)],
                sems.at[slot_],
            ).start()
            return 0
        lax.fori_loop(0, _RB, row, 0)

    def drain(block, slot_):
        def row(r, _):
            g = idx_ref[block * _RB + r]
            pltpu.make_async_copy(
                a_hbm.at[pl.ds(g, 1)],
                a_buf.at[slot_, pl.ds(r, 1)],
                sems.at[slot_],
            ).wait()
            return 0
        lax.fori_loop(0, _RB, row, 0)

    @pl.when(i == 0)
    def _prologue():
        issue(0, 0)
        t_scr[...] = lax.dot_general(
            w2t_ref[...], emb_ref[...], (((1,), (1,)), ((), ())),
            preferred_element_type=jnp.float32,
            precision=lax.Precision.HIGHEST)

    @pl.when(i + 1 < _NBLK)
    def _prefetch_next():
        issue(i + 1, 1 - slot)

    drain(i, slot)

    cen = cen_ref[...]
    c_col = jnp.dot(cen, w1_ref[...],
                    preferred_element_type=jnp.float32,
                    precision=lax.Precision.HIGHEST)
    s = c_col + t_scr[...] + b_ref[0, 0]
    s = jnp.where(s >= 0, s, 0.2 * s)
    w = jnp.where(a_buf[slot] > 0, s, 0.0)
    out_ref[...] = jnp.dot(w, emb_ref[...],
                           preferred_element_type=jnp.float32,
                           precision=lax.Precision.HIGHEST) + cen


def _tc_aggregate(node_indexes, a, emb, center, w1, w2t, b2d):
    grid_spec = pltpu.PrefetchScalarGridSpec(
        num_scalar_prefetch=1,
        grid=(_NBLK,),
        in_specs=[
            pl.BlockSpec(memory_space=pl.ANY),
            pl.BlockSpec((N, D), lambda i, idx: (0, 0)),
            pl.BlockSpec((_RB, D), lambda i, idx: (i, 0)),
            pl.BlockSpec((D, 1), lambda i, idx: (0, 0)),
            pl.BlockSpec((1, D), lambda i, idx: (0, 0)),
            pl.BlockSpec((1, 1), lambda i, idx: (0, 0)),
        ],
        out_specs=pl.BlockSpec((_RB, D), lambda i, idx: (i, 0)),
        scratch_shapes=[
            pltpu.VMEM((2, _RB, N), jnp.float32),
            pltpu.VMEM((1, N), jnp.float32),
            pltpu.SemaphoreType.DMA((2,)),
        ],
    )
    return pl.pallas_call(
        _tc_body,
        grid_spec=grid_spec,
        out_shape=jax.ShapeDtypeStruct((B, D), jnp.float32),
        compiler_params=pltpu.CompilerParams(
            vmem_limit_bytes=100 * 1024 * 1024),
    )(node_indexes, a, emb, center, w1, w2t, b2d)


def kernel(node_indexes, A, embedding_states, W_fc, b_fc):
    center = _sc_center(node_indexes, embedding_states)
    w1 = W_fc[:D]
    w2t = W_fc[D:].reshape(1, D)
    b2d = b_fc.reshape(1, 1)
    return _tc_aggregate(node_indexes, A, embedding_states, center, w1, w2t,
                         b2d)


# speedup vs baseline: 3.2464x; 3.2464x over previous
import jax
import jax.numpy as jnp
from jax import lax
from jax.experimental import pallas as pl
from jax.experimental.pallas import tpu as pltpu

N = 10000
D = 256
B = 1024

_RB = 128
_NBLK = B // _RB


def _body(idx_ref, a_hbm, emb_hbm, emb_ref, w1_ref, w2t_ref, b_ref,
          out_ref, a_buf, cen_buf, t_scr, sems):
    i = pl.program_id(0)
    slot = lax.rem(i, 2)

    def copies(block, slot_):
        descs = []
        for r in range(_RB):
            g = idx_ref[block * _RB + r]
            descs.append(pltpu.make_async_copy(
                a_hbm.at[pl.ds(g, 1)],
                a_buf.at[slot_, pl.ds(r, 1)],
                sems.at[slot_]))
            descs.append(pltpu.make_async_copy(
                emb_hbm.at[pl.ds(g, 1)],
                cen_buf.at[slot_, pl.ds(r, 1)],
                sems.at[slot_]))
        return descs

    def issue(block, slot_):
        for d in copies(block, slot_):
            d.start()

    def drain(block, slot_):
        for d in copies(block, slot_):
            d.wait()

    @pl.when(i == 0)
    def _prologue():
        issue(0, 0)
        t_scr[...] = lax.dot_general(
            w2t_ref[...], emb_ref[...], (((1,), (1,)), ((), ())),
            preferred_element_type=jnp.float32,
            precision=lax.Precision.HIGHEST) + b_ref[0, 0]

    @pl.when(i + 1 < _NBLK)
    def _prefetch_next():
        issue(i + 1, 1 - slot)

    drain(i, slot)

    cen = cen_buf[slot]
    c_col = jnp.dot(cen, w1_ref[...],
                    preferred_element_type=jnp.float32,
                    precision=lax.Precision.HIGHEST)
    s = c_col + t_scr[...]
    s = jnp.where(s >= 0, s, 0.2 * s)
    w = a_buf[slot] * s
    out_ref[...] = jnp.dot(w, emb_ref[...],
                           preferred_element_type=jnp.float32) + cen


def _aggregate(node_indexes, a, emb, w1, w2t, b2d):
    grid_spec = pltpu.PrefetchScalarGridSpec(
        num_scalar_prefetch=1,
        grid=(_NBLK,),
        in_specs=[
            pl.BlockSpec(memory_space=pl.ANY),
            pl.BlockSpec(memory_space=pl.ANY),
            pl.BlockSpec((N, D), lambda i, idx: (0, 0)),
            pl.BlockSpec((D, 1), lambda i, idx: (0, 0)),
            pl.BlockSpec((1, D), lambda i, idx: (0, 0)),
            pl.BlockSpec((1, 1), lambda i, idx: (0, 0)),
        ],
        out_specs=pl.BlockSpec((_RB, D), lambda i, idx: (i, 0)),
        scratch_shapes=[
            pltpu.VMEM((2, _RB, N), jnp.float32),
            pltpu.VMEM((2, _RB, D), jnp.float32),
            pltpu.VMEM((1, N), jnp.float32),
            pltpu.SemaphoreType.DMA((2,)),
        ],
    )
    return pl.pallas_call(
        _body,
        grid_spec=grid_spec,
        out_shape=jax.ShapeDtypeStruct((B, D), jnp.float32),
        compiler_params=pltpu.CompilerParams(
            vmem_limit_bytes=100 * 1024 * 1024),
    )(node_indexes, a, emb, emb, w1, w2t, b2d)


def kernel(node_indexes, A, embedding_states, W_fc, b_fc):
    w1 = W_fc[:D]
    w2t = W_fc[D:].reshape(1, D)
    b2d = b_fc.reshape(1, 1)
    return _aggregate(node_indexes, A, embedding_states, w1, w2t, b2d)


# speedup vs baseline: 4.1804x; 1.2877x over previous
import jax
import jax.numpy as jnp
from jax import lax
from jax.experimental import pallas as pl
from jax.experimental.pallas import tpu as pltpu

N = 10000
D = 256
B = 1024

_RB = 128
_NBLK = B // _RB


def _body(idx_ref, a_hbm, emb_hbm, emb_ref, w1_ref, w2t_ref, b_ref,
          out_ref, a_buf, cen_buf, t_scr, emb_bf, sems):
    i = pl.program_id(0)
    slot = lax.rem(i, 2)

    def copies(block, slot_):
        descs = []
        for r in range(_RB):
            g = idx_ref[block * _RB + r]
            descs.append(pltpu.make_async_copy(
                a_hbm.at[pl.ds(g, 1)],
                a_buf.at[slot_, pl.ds(r, 1)],
                sems.at[slot_]))
            descs.append(pltpu.make_async_copy(
                emb_hbm.at[pl.ds(g, 1)],
                cen_buf.at[slot_, pl.ds(r, 1)],
                sems.at[slot_]))
        return descs

    def issue(block, slot_):
        for d in copies(block, slot_):
            d.start()

    def drain(block, slot_):
        for d in copies(block, slot_):
            d.wait()

    @pl.when(i == 0)
    def _prologue():
        issue(0, 0)
        emb_bf[...] = emb_ref[...].astype(jnp.bfloat16)
        t_scr[...] = lax.dot_general(
            w2t_ref[...].astype(jnp.bfloat16), emb_bf[...],
            (((1,), (1,)), ((), ())),
            preferred_element_type=jnp.float32) + b_ref[0, 0]

    @pl.when(i + 1 < _NBLK)
    def _prefetch_next():
        issue(i + 1, 1 - slot)

    drain(i, slot)

    cen = cen_buf[slot]
    c_col = jnp.dot(cen, w1_ref[...],
                    preferred_element_type=jnp.float32)
    s = c_col + t_scr[...]
    s = jnp.where(s >= 0, s, 0.2 * s)
    w = (a_buf[slot] * s).astype(jnp.bfloat16)
    out_ref[...] = jnp.dot(w, emb_bf[...],
                           preferred_element_type=jnp.float32) + cen


def _aggregate(node_indexes, a, emb, w1, w2t, b2d):
    grid_spec = pltpu.PrefetchScalarGridSpec(
        num_scalar_prefetch=1,
        grid=(_NBLK,),
        in_specs=[
            pl.BlockSpec(memory_space=pl.ANY),
            pl.BlockSpec(memory_space=pl.ANY),
            pl.BlockSpec((N, D), lambda i, idx: (0, 0)),
            pl.BlockSpec((D, 1), lambda i, idx: (0, 0)),
            pl.BlockSpec((1, D), lambda i, idx: (0, 0)),
            pl.BlockSpec((1, 1), lambda i, idx: (0, 0)),
        ],
        out_specs=pl.BlockSpec((_RB, D), lambda i, idx: (i, 0)),
        scratch_shapes=[
            pltpu.VMEM((2, _RB, N), jnp.float32),
            pltpu.VMEM((2, _RB, D), jnp.float32),
            pltpu.VMEM((1, N), jnp.float32),
            pltpu.VMEM((N, D), jnp.bfloat16),
            pltpu.SemaphoreType.DMA((2,)),
        ],
    )
    return pl.pallas_call(
        _body,
        grid_spec=grid_spec,
        out_shape=jax.ShapeDtypeStruct((B, D), jnp.float32),
        compiler_params=pltpu.CompilerParams(
            vmem_limit_bytes=100 * 1024 * 1024),
    )(node_indexes, a, emb, emb, w1, w2t, b2d)


def kernel(node_indexes, A, embedding_states, W_fc, b_fc):
    w1 = W_fc[:D]
    w2t = W_fc[D:].reshape(1, D)
    b2d = b_fc.reshape(1, 1)
    return _aggregate(node_indexes, A, embedding_states, w1, w2t, b2d)
